# baseline (speedup 1.0000x reference)
import functools
import math

import jax
import jax.numpy as jnp
from jax import lax
from jax.experimental import pallas as pl
from jax.experimental.pallas import tpu as pltpu
from jax.experimental.pallas import tpu_sc as plsc

_VOCAB = 1000000
_EMB = 64
_B = 4096
_L = 200

_NC = 2
_NS = 16
_NW = _NC * _NS
_ROWS = _B // _NW
_NBUF = 6
_S0 = 104
_S1 = _L - _S0
_SCALE = math.sqrt(_EMB)
_RUNROLL = 8

_mesh = plsc.VectorSubcoreMesh(core_axis_name="c", subcore_axis_name="s")


@functools.partial(
    pl.kernel,
    mesh=_mesh,
    out_type=jax.ShapeDtypeStruct((_B, _L, _EMB), jnp.float32),
    scratch_types=(
        [pltpu.VMEM((_ROWS, _L), jnp.int32)]
        + [pltpu.VMEM((_L, _EMB), jnp.float32) for _ in range(_NBUF)]
        + [pltpu.SemaphoreType.DMA for _ in range(2 * _NBUF)]
    ),
    compiler_params=pltpu.CompilerParams(use_tc_tiling_on_sc=False),
)
def _embed(tok_hbm, table_hbm, out_hbm, idx_v, *bufs_and_sems):
    buf = bufs_and_sems[:_NBUF]
    gsem = bufs_and_sems[_NBUF:2 * _NBUF]
    ssem = bufs_and_sems[2 * _NBUF:3 * _NBUF]

    wid = lax.axis_index("s") * _NC + lax.axis_index("c")
    base = wid * _ROWS
    pltpu.sync_copy(tok_hbm.at[pl.ds(base, _ROWS)], idx_v)

    def fire_gather(r, b):
        pltpu.async_copy(
            table_hbm.at[idx_v.at[r, pl.ds(0, _S0)]],
            buf[b].at[pl.ds(0, _S0)], gsem[b])
        pltpu.async_copy(
            table_hbm.at[idx_v.at[r, pl.ds(_S0, _S1)]],
            buf[b].at[pl.ds(_S0, _S1)], gsem[b])

    def wait_gather(r, b):
        pltpu.make_async_copy(
            table_hbm.at[idx_v.at[r, pl.ds(0, _S0)]],
            buf[b].at[pl.ds(0, _S0)], gsem[b]).wait()
        pltpu.make_async_copy(
            table_hbm.at[idx_v.at[r, pl.ds(_S0, _S1)]],
            buf[b].at[pl.ds(_S0, _S1)], gsem[b]).wait()

    def wait_store(r, b):
        pltpu.make_async_copy(buf[b], out_hbm.at[base + r], ssem[b]).wait()

    for b in range(_NBUF - 1):
        fire_gather(b, b)

    def scale_buf(b):
        def step(t, carry):
            for rr in range(_RUNROLL):
                r = t * _RUNROLL + rr
                for c in range(_EMB // 16):
                    sl = (r, pl.ds(c * 16, 16))
                    buf[b][sl] = buf[b][sl] * _SCALE
            return carry

        lax.fori_loop(0, _L // _RUNROLL, step, None)

    def outer(it, carry):
        r0 = it * _NBUF
        for b in range(_NBUF):
            r = r0 + b
            bp = (b - 1) % _NBUF
            wait_gather(r, b)
            scale_buf(b)
            pltpu.async_copy(buf[b], out_hbm.at[base + r], ssem[b])

            @pl.when(r > 0)
            def _():
                wait_store(r - 1, bp)

            @pl.when(r + _NBUF - 1 < _ROWS)
            def _():
                fire_gather(r + _NBUF - 1, bp)

        return carry

    _FULL = _ROWS // _NBUF
    lax.fori_loop(0, _FULL, outer, None)
    for r in range(_FULL * _NBUF, _ROWS):
        b = r % _NBUF
        bp = (b - 1) % _NBUF
        wait_gather(r, b)
        scale_buf(b)
        pltpu.async_copy(buf[b], out_hbm.at[base + r], ssem[b])
        wait_store(r - 1, bp)

    wait_store(_ROWS - 1, (_ROWS - 1) % _NBUF)


def kernel(tokens, table):
    out = _embed(tokens.astype(jnp.int32), table)
    return out
